# baseline (speedup 1.0000x reference)
import functools

import jax
import jax.numpy as jnp
from jax import lax
from jax.experimental import pallas as pl
from jax.experimental.pallas import tpu as pltpu
from jax.experimental.pallas import tpu_sc as plsc


_BS = 2048


def _tc_body(x_ref, pos_ref, out_ref):
    out_ref[0, :, :] = x_ref[0, :, :] + pos_ref[:, :]


def _kernel_tc(x, pos_table):
    batch, seq_len, d_model = x.shape
    nb = seq_len // _BS
    return pl.pallas_call(
        _tc_body,
        grid=(nb, batch),
        in_specs=[
            pl.BlockSpec((1, _BS, d_model), lambda i, j: (j, i, 0)),
            pl.BlockSpec((_BS, d_model), lambda i, j: (i, 0)),
        ],
        out_specs=pl.BlockSpec((1, _BS, d_model), lambda i, j: (j, i, 0)),
        out_shape=jax.ShapeDtypeStruct(x.shape, x.dtype),
    )(x, pos_table[:seq_len])


_NC = 2
_NS = 16
_NW = _NC * _NS

_D = 1024
_SEQ = 2048
_BATCH = 4
_ROWS_PER_W = _SEQ // _NW
_CHUNK_ROWS = 16
_CHUNK = _CHUNK_ROWS * _D
_NCHUNK_PER_B = _ROWS_PER_W // _CHUNK_ROWS
_STRIP = _ROWS_PER_W * _D
_LANES = 16


_NXBUF = 4


def _sc_body(x_hbm, pos_hbm, out_hbm, pv0, pv1, xb0, xb1, xb2, xb3,
             sp0, sp1, si0, si1, si2, si3, so0, so1, so2, so3):
    wid = lax.axis_index("s") * _NC + lax.axis_index("c")
    row0 = wid * _ROWS_PER_W

    pos_bufs = (pv0, pv1)
    pos_sems = (sp0, sp1)
    bufs = (xb0, xb1, xb2, xb3)
    in_sems = (si0, si1, si2, si3)
    out_sems = (so0, so1, so2, so3)

    chunks = []
    for c in range(_NCHUNK_PER_B):
        for b in range(_BATCH):
            chunks.append((b, c))
    n = len(chunks)

    def x_slice(i):
        b, c = chunks[i]
        return pl.ds(b * _SEQ + row0 + c * _CHUNK_ROWS, _CHUNK_ROWS)

    def pos_slice(c):
        return pl.ds(row0 + c * _CHUNK_ROWS, _CHUNK_ROWS)

    in_copy = [None] * _NXBUF
    out_copies = [None] * _NXBUF
    pos_copy = [None, None]

    pos_copy[0] = pltpu.async_copy(pos_hbm.at[pos_slice(0)], pos_bufs[0], pos_sems[0])
    for j in range(_NXBUF - 1):
        in_copy[j] = pltpu.async_copy(x_hbm.at[x_slice(j)], bufs[j], in_sems[j])

    pos_waited = [False] * _NCHUNK_PER_B

    for i in range(n):
        k = i % _NXBUF
        b, c = chunks[i]
        if i + _NXBUF - 1 < n:
            j = (i + _NXBUF - 1) % _NXBUF
            if out_copies[j] is not None:
                out_copies[j].wait()
                out_copies[j] = None
            in_copy[j] = pltpu.async_copy(
                x_hbm.at[x_slice(i + _NXBUF - 1)], bufs[j], in_sems[j]
            )
        if b == 0 and c + 1 < _NCHUNK_PER_B:
            pos_copy[(c + 1) % 2] = pltpu.async_copy(
                pos_hbm.at[pos_slice(c + 1)], pos_bufs[(c + 1) % 2], pos_sems[(c + 1) % 2]
            )
        if not pos_waited[c]:
            pos_copy[c % 2].wait()
            pos_waited[c] = True
        in_copy[k].wait()
        if out_copies[k] is not None:
            out_copies[k].wait()
            out_copies[k] = None

        xb = bufs[k]
        pv = pos_bufs[c % 2]

        @plsc.parallel_loop(0, _CHUNK, _LANES, unroll=8)
        def _add(off, xb=xb, pv=pv):
            r = lax.shift_right_logical(off, 10)
            cc = pl.multiple_of(lax.bitwise_and(off, _D - 1), _LANES)
            plsc.addupdate(xb.at[r, pl.ds(cc, _LANES)], pv[r, pl.ds(cc, _LANES)])

        out_copies[k] = pltpu.async_copy(xb, out_hbm.at[x_slice(i)], out_sems[k])

    for oc in out_copies:
        if oc is not None:
            oc.wait()


def _kernel_sc(x, pos_table):
    batch, seq_len, d_model = x.shape
    x2 = x.reshape(batch * seq_len, d_model)
    mesh = plsc.VectorSubcoreMesh(core_axis_name="c", subcore_axis_name="s")
    out2 = pl.kernel(
        _sc_body,
        out_type=jax.ShapeDtypeStruct((batch * seq_len, d_model), jnp.float32),
        mesh=mesh,
        scratch_types=[
            pltpu.VMEM((_CHUNK_ROWS, _D), jnp.float32),
            pltpu.VMEM((_CHUNK_ROWS, _D), jnp.float32),
            pltpu.VMEM((_CHUNK_ROWS, _D), jnp.float32),
            pltpu.VMEM((_CHUNK_ROWS, _D), jnp.float32),
            pltpu.VMEM((_CHUNK_ROWS, _D), jnp.float32),
            pltpu.VMEM((_CHUNK_ROWS, _D), jnp.float32),
            pltpu.SemaphoreType.DMA,
            pltpu.SemaphoreType.DMA,
            pltpu.SemaphoreType.DMA,
            pltpu.SemaphoreType.DMA,
            pltpu.SemaphoreType.DMA,
            pltpu.SemaphoreType.DMA,
            pltpu.SemaphoreType.DMA,
            pltpu.SemaphoreType.DMA,
            pltpu.SemaphoreType.DMA,
            pltpu.SemaphoreType.DMA,
        ],
    )(x2, pos_table)
    return out2.reshape(x.shape)


def kernel(x, pos_table):
    return _kernel_sc(x, pos_table)
